# Initial kernel scaffold; baseline (speedup 1.0000x reference)
import jax
import jax.numpy as jnp
from jax.experimental import pallas as pl


def kernel(features, knn_indices, W0, b0, W1, b1, W2, b2):
    raise NotImplementedError("write your pallas kernel here")


# speedup vs baseline: 3.7544x; 3.7544x over previous
import functools

import jax
import jax.numpy as jnp
from jax import lax
from jax.experimental import pallas as pl
from jax.experimental.pallas import tpu as pltpu
from jax.experimental.pallas import tpu_sc as plsc

NW = 32
BC = 112
BN = 512


def _mm_body(x_ref, w_ref, b_ref, o_ref):
    o_ref[...] = (
        jnp.dot(x_ref[...], w_ref[...], preferred_element_type=jnp.float32)
        + b_ref[...]
    )


def _matmul_bias(x, wt, bfull):
    npad, cin = x.shape
    koc = wt.shape[1]
    nb = npad // BN
    return pl.pallas_call(
        _mm_body,
        grid=(nb,),
        in_specs=[
            pl.BlockSpec((BN, cin), lambda i: (i, 0)),
            pl.BlockSpec((cin, koc), lambda i: (0, 0)),
            pl.BlockSpec((1, koc), lambda i: (0, 0)),
        ],
        out_specs=pl.BlockSpec((BN, koc), lambda i: (i, 0)),
        out_shape=jax.ShapeDtypeStruct((npad, koc), jnp.float32),
        compiler_params=pltpu.CompilerParams(
            dimension_semantics=("parallel",)
        ),
    )(x, wt, bfull)


def _gather_sum(idxc, table, oc, act, npad):
    nch = idxc.shape[1]
    kk = idxc.shape[2]
    mesh = plsc.VectorSubcoreMesh(core_axis_name="c", subcore_axis_name="s")

    @functools.partial(
        pl.kernel,
        out_type=jax.ShapeDtypeStruct((npad, oc), jnp.float32),
        mesh=mesh,
        scratch_types=(
            [pltpu.VMEM((kk, BC), jnp.int32)]
            + [pltpu.VMEM((BC, oc), jnp.float32) for _ in range(kk)]
            + [pltpu.VMEM((BC, oc), jnp.float32), pltpu.SemaphoreType.DMA]
        ),
        compiler_params=pltpu.CompilerParams(use_tc_tiling_on_sc=False),
    )
    def run(idx_hbm, table_hbm, out_hbm, idx_v, *rest):
        bufs = rest[:kk]
        out_v = rest[kk]
        sem = rest[kk + 1]
        wid = lax.axis_index("s") * 2 + lax.axis_index("c")
        base0 = wid * (nch * BC)
        for c in range(nch):
            base = base0 + c * BC
            pltpu.sync_copy(idx_hbm.at[wid, c], idx_v)
            cps = [
                pltpu.async_copy(table_hbm.at[idx_v.at[j]], bufs[j], sem)
                for j in range(kk)
            ]
            for cp in cps:
                cp.wait()

            def row(r, carry):
                for c2 in range(oc // 16):
                    sl = pl.ds(c2 * 16, 16)
                    s = bufs[0][r, sl]
                    for j in range(1, kk):
                        s = s + bufs[j][r, sl]
                    if act:
                        s = 1.0 / (1.0 + jnp.exp(-s))
                    out_v[r, sl] = s
                return carry

            lax.fori_loop(0, BC, row, 0)
            pltpu.sync_copy(out_v, out_hbm.at[pl.ds(base, BC)])

    return run(idxc, table)


def kernel(features, knn_indices, W0, b0, W1, b1, W2, b2):
    n, _ = features.shape
    k = knn_indices.shape[1]
    nch = -(-n // (NW * BC))
    npad = NW * BC * nch

    x = jnp.pad(features, ((0, npad - n), (0, 0)))
    idx = jnp.pad(knn_indices, ((0, npad - n), (0, 0)))
    idxa = idx * k + jnp.arange(k, dtype=jnp.int32)[None, :]
    idxc = idxa.reshape(NW, nch, BC, k).transpose(0, 1, 3, 2)

    w2p = jnp.pad(W2, ((0, 16 - W2.shape[0]), (0, 0)))
    b2p = jnp.pad(b2, ((0, 0), (0, 16 - b2.shape[1])))

    h = x
    for wgt, bias, act in ((W0, b0, True), (W1, b1, True), (w2p, b2p, False)):
        oc = wgt.shape[0]
        cin = h.shape[1]
        wt = wgt.reshape(oc, k, cin).transpose(1, 0, 2).reshape(k * oc, cin).T
        bfull = jnp.pad(bias, ((0, 0), (0, (k - 1) * oc)))
        y = _matmul_bias(h, wt, bfull)
        table = y.reshape(npad * k, oc)
        h = _gather_sum(idxc, table, oc, act, npad)

    return h[:n, :3]
